# baseline (speedup 1.0000x reference)
import jax
import jax.numpy as jnp
from jax.experimental import pallas as pl


def _add_pos_kernel(x_ref, ph_ref, pw_ref, o_ref):
    ph = ph_ref[...]
    pw = pw_ref[...]
    o_ref[...] = x_ref[...] + ph[None, :, None, :] + pw[None, None, :, :]


def kernel(x, pos_height, pos_width):
    B, H, W, D = x.shape
    NB = 1
    return pl.pallas_call(
        _add_pos_kernel,
        grid=(B // NB,),
        in_specs=[
            pl.BlockSpec((NB, H, W, D), lambda b: (b, 0, 0, 0)),
            pl.BlockSpec((H, D), lambda b: (0, 0)),
            pl.BlockSpec((W, D), lambda b: (0, 0)),
        ],
        out_specs=pl.BlockSpec((NB, H, W, D), lambda b: (b, 0, 0, 0)),
        out_shape=jax.ShapeDtypeStruct((B, H, W, D), x.dtype),
    )(x, pos_height, pos_width)


# speedup vs baseline: 1.0333x; 1.0333x over previous
import jax
import jax.numpy as jnp
from jax.experimental import pallas as pl


def _add_pos_kernel(x_ref, ph_ref, pw_ref, o_ref):
    ph = ph_ref[...]
    pw = pw_ref[...]
    o_ref[...] = x_ref[...] + ph[None, :, None, :] + pw[None, None, :, :]


def kernel(x, pos_height, pos_width):
    B, H, W, D = x.shape
    NB = 2
    return pl.pallas_call(
        _add_pos_kernel,
        grid=(B // NB,),
        in_specs=[
            pl.BlockSpec((NB, H, W, D), lambda b: (b, 0, 0, 0)),
            pl.BlockSpec((H, D), lambda b: (0, 0)),
            pl.BlockSpec((W, D), lambda b: (0, 0)),
        ],
        out_specs=pl.BlockSpec((NB, H, W, D), lambda b: (b, 0, 0, 0)),
        out_shape=jax.ShapeDtypeStruct((B, H, W, D), x.dtype),
    )(x, pos_height, pos_width)
